# baseline (speedup 1.0000x reference)
import jax
import jax.numpy as jnp
from jax.experimental import pallas as pl
from jax.experimental.pallas import tpu as pltpu

BS, L, DIM = 4, 512, 128
IG = 128
NG = L // IG
CH = 8
NCH = IG // CH


def _gcn_body(q_ref, qg_ref, qT_ref, qgT_ref, depT_ref, a_ref, b_ref,
              w2bd_ref, wdT_ref, outT_ref, prei_ref, prej_ref, h8a_ref, h8b_ref,
              gacc_ref, hjT_ref):
    @pl.when(pl.program_id(1) == 0)
    def _():
        prej_ref[...] = jnp.dot(q_ref[0], b_ref[...],
                                preferred_element_type=jnp.float32)
        hjT_ref[...] = jnp.dot(wdT_ref[...], qT_ref[0],
                               preferred_element_type=jnp.float32)

    prei_ref[...] = jnp.dot(qg_ref[0], a_ref[...],
                            preferred_element_type=jnp.float32)
    gacc_ref[...] = jnp.zeros((L, IG), jnp.float32)

    u_iota = jax.lax.broadcasted_iota(jnp.int32, (CH, IG), 0)
    l_iota = jax.lax.broadcasted_iota(jnp.int32, (CH, IG), 1)

    def build(buf_ref, k):
        prej = prej_ref[...]
        for u in range(CH):
            r = prei_ref[pl.ds(k * CH + u, 1), :]
            buf_ref[:, DIM * u:DIM * (u + 1)] = (
                jnp.maximum(prej + r, 0.0).astype(jnp.bfloat16))

    def reduce(buf_ref, k):
        tmp = jnp.dot(buf_ref[...], w2bd_ref[...],
                      preferred_element_type=jnp.float32)
        place = (l_iota == CH * k + u_iota).astype(jnp.float32)
        gacc_ref[...] += jnp.dot(tmp, place,
                                 preferred_element_type=jnp.float32)

    build(h8a_ref, 0)

    def step(k2, _):
        build(h8b_ref, 2 * k2 + 1)
        reduce(h8a_ref, 2 * k2)
        build(h8a_ref, jax.lax.rem(2 * k2 + 2, NCH))
        reduce(h8b_ref, 2 * k2 + 1)
        return 0

    jax.lax.fori_loop(0, NCH // 2, step, 0)

    maskT = depT_ref[0] > 0
    T = jnp.where(maskT, gacc_ref[...], jnp.float32(-100.0))
    m = jnp.max(T, axis=0, keepdims=True)
    e = jnp.exp(T - m)
    r = 1.0 / jnp.sum(e, axis=0, keepdims=True)
    betam = e * r * maskT.astype(jnp.float32)

    aggT = jnp.dot(hjT_ref[...], betam,
                   preferred_element_type=jnp.float32)
    outT_ref[0] = jnp.maximum(qgT_ref[0] + aggT, 0.0)


def kernel(queries, wordlens, syntactic_dep, W1, W2, Wd):
    q = queries.astype(jnp.float32)
    qT = jnp.swapaxes(q, 1, 2)
    depT = jnp.swapaxes(syntactic_dep.astype(jnp.int32), 1, 2)
    A = W1[:DIM, :]
    B = W1[DIM:, :]
    W2bd = jnp.kron(jnp.eye(CH, dtype=jnp.float32),
                    W2).astype(jnp.bfloat16)
    WdT = jnp.swapaxes(Wd, 0, 1)

    outT = pl.pallas_call(
        _gcn_body,
        grid=(BS, NG),
        in_specs=[
            pl.BlockSpec((1, L, DIM), lambda b, g: (b, 0, 0)),
            pl.BlockSpec((1, IG, DIM), lambda b, g: (b, g, 0)),
            pl.BlockSpec((1, DIM, L), lambda b, g: (b, 0, 0)),
            pl.BlockSpec((1, DIM, IG), lambda b, g: (b, 0, g)),
            pl.BlockSpec((1, L, IG), lambda b, g: (b, 0, g)),
            pl.BlockSpec((DIM, DIM), lambda b, g: (0, 0)),
            pl.BlockSpec((DIM, DIM), lambda b, g: (0, 0)),
            pl.BlockSpec((CH * DIM, CH), lambda b, g: (0, 0)),
            pl.BlockSpec((DIM, DIM), lambda b, g: (0, 0)),
        ],
        out_specs=pl.BlockSpec((1, DIM, IG), lambda b, g: (b, 0, g)),
        out_shape=jax.ShapeDtypeStruct((BS, DIM, L), jnp.float32),
        scratch_shapes=[
            pltpu.VMEM((IG, DIM), jnp.float32),
            pltpu.VMEM((L, DIM), jnp.float32),
            pltpu.VMEM((L, CH * DIM), jnp.bfloat16),
            pltpu.VMEM((L, CH * DIM), jnp.bfloat16),
            pltpu.VMEM((L, IG), jnp.float32),
            pltpu.VMEM((DIM, L), jnp.float32),
        ],
        compiler_params=pltpu.CompilerParams(
            dimension_semantics=("arbitrary", "arbitrary"),
        ),
    )(q, q, qT, qT, depT, A, B, W2bd, WdT)

    out = jnp.swapaxes(outT, 1, 2)
    return (out, wordlens, syntactic_dep)


# speedup vs baseline: 1.1848x; 1.1848x over previous
import jax
import jax.numpy as jnp
from jax.experimental import pallas as pl
from jax.experimental.pallas import tpu as pltpu

BS, L, DIM = 4, 512, 128
IG = 128
NG = L // IG
CH = 8
NCH = IG // CH


def _gcn_body(q_ref, qg_ref, qT_ref, qgT_ref, depT_ref, a_ref, b_ref,
              w2bd_ref, wdT_ref, outT_ref, prei_ref, prej_ref, h8a_ref, h8b_ref,
              gacc_ref, hjT_ref):
    @pl.when(pl.program_id(1) == 0)
    def _():
        prej_ref[...] = jnp.dot(q_ref[0], b_ref[...],
                                preferred_element_type=jnp.float32)
        hjT_ref[...] = jnp.dot(wdT_ref[...], qT_ref[0],
                               preferred_element_type=jnp.float32)

    prei_ref[...] = jnp.dot(qg_ref[0], a_ref[...],
                            preferred_element_type=jnp.float32)
    gacc_ref[...] = jnp.zeros((L, IG), jnp.float32)

    u_iota = jax.lax.broadcasted_iota(jnp.int32, (CH, IG), 0)
    l_iota = jax.lax.broadcasted_iota(jnp.int32, (CH, IG), 1)

    def build(buf_ref, k):
        prej = prej_ref[...]
        for u in range(CH):
            r = prei_ref[pl.ds(k * CH + u, 1), :]
            buf_ref[:, DIM * u:DIM * (u + 1)] = (
                jnp.maximum(prej + r, 0.0))

    def reduce(buf_ref, k):
        tmp = jnp.dot(buf_ref[...], w2bd_ref[...],
                      preferred_element_type=jnp.float32)
        place = (l_iota == CH * k + u_iota).astype(jnp.float32)
        gacc_ref[...] += jnp.dot(tmp, place,
                                 preferred_element_type=jnp.float32)

    build(h8a_ref, 0)

    def step(k2, _):
        build(h8b_ref, 2 * k2 + 1)
        reduce(h8a_ref, 2 * k2)
        build(h8a_ref, jax.lax.rem(2 * k2 + 2, NCH))
        reduce(h8b_ref, 2 * k2 + 1)
        return 0

    jax.lax.fori_loop(0, NCH // 2, step, 0)

    maskT = depT_ref[0] > 0
    T = jnp.where(maskT, gacc_ref[...], jnp.float32(-100.0))
    m = jnp.max(T, axis=0, keepdims=True)
    e = jnp.exp(T - m)
    r = 1.0 / jnp.sum(e, axis=0, keepdims=True)
    betam = e * r * maskT.astype(jnp.float32)

    aggT = jnp.dot(hjT_ref[...], betam,
                   preferred_element_type=jnp.float32)
    outT_ref[0] = jnp.maximum(qgT_ref[0] + aggT, 0.0)


def kernel(queries, wordlens, syntactic_dep, W1, W2, Wd):
    q = queries.astype(jnp.float32)
    qT = jnp.swapaxes(q, 1, 2)
    depT = jnp.swapaxes(syntactic_dep.astype(jnp.int32), 1, 2)
    A = W1[:DIM, :]
    B = W1[DIM:, :]
    W2bd = jnp.kron(jnp.eye(CH, dtype=jnp.float32), W2)
    WdT = jnp.swapaxes(Wd, 0, 1)

    outT = pl.pallas_call(
        _gcn_body,
        grid=(BS, NG),
        in_specs=[
            pl.BlockSpec((1, L, DIM), lambda b, g: (b, 0, 0)),
            pl.BlockSpec((1, IG, DIM), lambda b, g: (b, g, 0)),
            pl.BlockSpec((1, DIM, L), lambda b, g: (b, 0, 0)),
            pl.BlockSpec((1, DIM, IG), lambda b, g: (b, 0, g)),
            pl.BlockSpec((1, L, IG), lambda b, g: (b, 0, g)),
            pl.BlockSpec((DIM, DIM), lambda b, g: (0, 0)),
            pl.BlockSpec((DIM, DIM), lambda b, g: (0, 0)),
            pl.BlockSpec((CH * DIM, CH), lambda b, g: (0, 0)),
            pl.BlockSpec((DIM, DIM), lambda b, g: (0, 0)),
        ],
        out_specs=pl.BlockSpec((1, DIM, IG), lambda b, g: (b, 0, g)),
        out_shape=jax.ShapeDtypeStruct((BS, DIM, L), jnp.float32),
        scratch_shapes=[
            pltpu.VMEM((IG, DIM), jnp.float32),
            pltpu.VMEM((L, DIM), jnp.float32),
            pltpu.VMEM((L, CH * DIM), jnp.float32),
            pltpu.VMEM((L, CH * DIM), jnp.float32),
            pltpu.VMEM((L, IG), jnp.float32),
            pltpu.VMEM((DIM, L), jnp.float32),
        ],
        compiler_params=pltpu.CompilerParams(
            dimension_semantics=("arbitrary", "arbitrary"),
        ),
    )(q, q, qT, qT, depT, A, B, W2bd, WdT)

    out = jnp.swapaxes(outT, 1, 2)
    return (out, wordlens, syntactic_dep)


# speedup vs baseline: 1.2311x; 1.0391x over previous
import jax
import jax.numpy as jnp
from jax.experimental import pallas as pl
from jax.experimental.pallas import tpu as pltpu

BS, L, DIM = 4, 512, 128
IG = 128
NG = L // IG
CH = 8
NCH = IG // CH


def _gcn_body(q_ref, qg_ref, qT_ref, qgT_ref, depT_ref, a_ref, b_ref,
              w2bd_ref, wdT_ref, outT_ref, prei_ref, prej_ref, h8a_ref, h8b_ref,
              gacc_ref, hjT_ref):
    @pl.when(pl.program_id(1) == 0)
    def _():
        prej_ref[...] = jnp.dot(q_ref[0], b_ref[...],
                                preferred_element_type=jnp.float32)
        hjT_ref[...] = jnp.dot(wdT_ref[...], qT_ref[0],
                               preferred_element_type=jnp.float32)

    prei_ref[...] = jnp.dot(qg_ref[0], a_ref[...],
                            preferred_element_type=jnp.float32)
    gacc_ref[...] = jnp.zeros((L, IG), jnp.float32)

    u_iota = jax.lax.broadcasted_iota(jnp.int32, (CH, IG), 0)
    l_iota = jax.lax.broadcasted_iota(jnp.int32, (CH, IG), 1)

    def build(buf_ref, k):
        prej = prej_ref[...]
        for u in range(CH):
            r = prei_ref[pl.ds(k * CH + u, 1), :]
            buf_ref[:, DIM * u:DIM * (u + 1)] = (
                jnp.maximum(prej + r, 0.0))

    def reduce(buf_ref, k):
        tmp = jnp.dot(buf_ref[...], w2bd_ref[...],
                      preferred_element_type=jnp.float32)
        place = (l_iota == CH * k + u_iota).astype(jnp.float32)
        gacc_ref[...] += jnp.dot(tmp, place,
                                 preferred_element_type=jnp.float32)

    def step(k2, _):
        build(h8a_ref, 2 * k2)
        reduce(h8a_ref, 2 * k2)
        build(h8b_ref, 2 * k2 + 1)
        reduce(h8b_ref, 2 * k2 + 1)
        return 0

    jax.lax.fori_loop(0, NCH // 2, step, 0)

    maskT = depT_ref[0] > 0
    T = jnp.where(maskT, gacc_ref[...], jnp.float32(-100.0))
    m = jnp.max(T, axis=0, keepdims=True)
    e = jnp.exp(T - m)
    r = 1.0 / jnp.sum(e, axis=0, keepdims=True)
    betam = e * r * maskT.astype(jnp.float32)

    aggT = jnp.dot(hjT_ref[...], betam,
                   preferred_element_type=jnp.float32)
    outT_ref[0] = jnp.maximum(qgT_ref[0] + aggT, 0.0)


def kernel(queries, wordlens, syntactic_dep, W1, W2, Wd):
    q = queries.astype(jnp.float32)
    qT = jnp.swapaxes(q, 1, 2)
    depT = jnp.swapaxes(syntactic_dep.astype(jnp.int32), 1, 2)
    A = W1[:DIM, :]
    B = W1[DIM:, :]
    W2bd = jnp.kron(jnp.eye(CH, dtype=jnp.float32), W2)
    WdT = jnp.swapaxes(Wd, 0, 1)

    outT = pl.pallas_call(
        _gcn_body,
        grid=(BS, NG),
        in_specs=[
            pl.BlockSpec((1, L, DIM), lambda b, g: (b, 0, 0)),
            pl.BlockSpec((1, IG, DIM), lambda b, g: (b, g, 0)),
            pl.BlockSpec((1, DIM, L), lambda b, g: (b, 0, 0)),
            pl.BlockSpec((1, DIM, IG), lambda b, g: (b, 0, g)),
            pl.BlockSpec((1, L, IG), lambda b, g: (b, 0, g)),
            pl.BlockSpec((DIM, DIM), lambda b, g: (0, 0)),
            pl.BlockSpec((DIM, DIM), lambda b, g: (0, 0)),
            pl.BlockSpec((CH * DIM, CH), lambda b, g: (0, 0)),
            pl.BlockSpec((DIM, DIM), lambda b, g: (0, 0)),
        ],
        out_specs=pl.BlockSpec((1, DIM, IG), lambda b, g: (b, 0, g)),
        out_shape=jax.ShapeDtypeStruct((BS, DIM, L), jnp.float32),
        scratch_shapes=[
            pltpu.VMEM((IG, DIM), jnp.float32),
            pltpu.VMEM((L, DIM), jnp.float32),
            pltpu.VMEM((L, CH * DIM), jnp.float32),
            pltpu.VMEM((L, CH * DIM), jnp.float32),
            pltpu.VMEM((L, IG), jnp.float32),
            pltpu.VMEM((DIM, L), jnp.float32),
        ],
        compiler_params=pltpu.CompilerParams(
            dimension_semantics=("arbitrary", "arbitrary"),
        ),
    )(q, q, qT, qT, depT, A, B, W2bd, WdT)

    out = jnp.swapaxes(outT, 1, 2)
    return (out, wordlens, syntactic_dep)


# speedup vs baseline: 1.3552x; 1.1007x over previous
import jax
import jax.numpy as jnp
from jax.experimental import pallas as pl
from jax.experimental.pallas import tpu as pltpu

BS, L, DIM = 4, 512, 128
IG = 128
NG = L // IG
CH = 8
NCH = IG // CH


def _gcn_body(q_ref, qg_ref, qT_ref, qgT_ref, depT_ref, a_ref, b_ref,
              w2bd_ref, wdT_ref, outT_ref, prei_ref, prej_ref, h8_ref,
              gacc_ref):
    prei_ref[...] = jnp.dot(qg_ref[0], a_ref[...],
                            preferred_element_type=jnp.float32)
    prej_ref[...] = jnp.dot(q_ref[0], b_ref[...],
                            preferred_element_type=jnp.float32)
    gacc_ref[...] = jnp.zeros((L, IG), jnp.float32)

    u_iota = jax.lax.broadcasted_iota(jnp.int32, (CH, IG), 0)
    l_iota = jax.lax.broadcasted_iota(jnp.int32, (CH, IG), 1)

    def chunk(k, _):
        buf = jax.lax.rem(k, 2)
        prej = prej_ref[...]
        for u in range(CH):
            r = prei_ref[pl.ds(k * CH + u, 1), :]
            h8_ref[buf, :, DIM * u:DIM * (u + 1)] = (
                jnp.maximum(prej + r, 0.0))
        tmp = jnp.dot(h8_ref[buf], w2bd_ref[...],
                      preferred_element_type=jnp.float32)
        place = (l_iota == CH * k + u_iota).astype(jnp.float32)
        gacc_ref[...] += jnp.dot(tmp, place,
                                 preferred_element_type=jnp.float32)
        return 0

    jax.lax.fori_loop(0, NCH, chunk, 0)

    maskT = depT_ref[0] > 0
    T = jnp.where(maskT, gacc_ref[...], jnp.float32(-100.0))
    m = jnp.max(T, axis=0, keepdims=True)
    e = jnp.exp(T - m)
    betam = e / jnp.sum(e, axis=0, keepdims=True) * maskT.astype(jnp.float32)

    HjT = jnp.dot(wdT_ref[...], qT_ref[0],
                  preferred_element_type=jnp.float32)
    aggT = jnp.dot(HjT, betam, preferred_element_type=jnp.float32)
    outT_ref[0] = jnp.maximum(qgT_ref[0] + aggT, 0.0)


def kernel(queries, wordlens, syntactic_dep, W1, W2, Wd):
    q = queries.astype(jnp.float32)
    qT = jnp.swapaxes(q, 1, 2)
    depT = jnp.swapaxes(syntactic_dep.astype(jnp.int32), 1, 2)
    A = W1[:DIM, :]
    B = W1[DIM:, :]
    W2bd = jnp.kron(jnp.eye(CH, dtype=jnp.float32), W2)
    WdT = jnp.swapaxes(Wd, 0, 1)

    outT = pl.pallas_call(
        _gcn_body,
        grid=(BS, NG),
        in_specs=[
            pl.BlockSpec((1, L, DIM), lambda b, g: (b, 0, 0)),
            pl.BlockSpec((1, IG, DIM), lambda b, g: (b, g, 0)),
            pl.BlockSpec((1, DIM, L), lambda b, g: (b, 0, 0)),
            pl.BlockSpec((1, DIM, IG), lambda b, g: (b, 0, g)),
            pl.BlockSpec((1, L, IG), lambda b, g: (b, 0, g)),
            pl.BlockSpec((DIM, DIM), lambda b, g: (0, 0)),
            pl.BlockSpec((DIM, DIM), lambda b, g: (0, 0)),
            pl.BlockSpec((CH * DIM, CH), lambda b, g: (0, 0)),
            pl.BlockSpec((DIM, DIM), lambda b, g: (0, 0)),
        ],
        out_specs=pl.BlockSpec((1, DIM, IG), lambda b, g: (b, 0, g)),
        out_shape=jax.ShapeDtypeStruct((BS, DIM, L), jnp.float32),
        scratch_shapes=[
            pltpu.VMEM((IG, DIM), jnp.float32),
            pltpu.VMEM((L, DIM), jnp.float32),
            pltpu.VMEM((2, L, CH * DIM), jnp.float32),
            pltpu.VMEM((L, IG), jnp.float32),
        ],
        compiler_params=pltpu.CompilerParams(
            dimension_semantics=("arbitrary", "arbitrary"),
        ),
    )(q, q, qT, qT, depT, A, B, W2bd, WdT)

    out = jnp.swapaxes(outT, 1, 2)
    return (out, wordlens, syntactic_dep)


# speedup vs baseline: 1.3569x; 1.0013x over previous
import jax
import jax.numpy as jnp
from jax.experimental import pallas as pl
from jax.experimental.pallas import tpu as pltpu

BS, L, DIM = 4, 512, 128
IG = 128
NG = L // IG
CH = 8
NCH = IG // CH


def _gcn_body(q_ref, qg_ref, qT_ref, qgT_ref, depT_ref, a_ref, b_ref,
              w2bd_ref, wdT_ref, outT_ref, prei_ref, prej_ref,
              gacc_ref):
    prei_ref[...] = jnp.dot(qg_ref[0], a_ref[...],
                            preferred_element_type=jnp.float32)
    prej_ref[...] = jnp.dot(q_ref[0], b_ref[...],
                            preferred_element_type=jnp.float32)
    gacc_ref[...] = jnp.zeros((L, IG), jnp.float32)

    u_iota = jax.lax.broadcasted_iota(jnp.int32, (CH, IG), 0)
    l_iota = jax.lax.broadcasted_iota(jnp.int32, (CH, IG), 1)

    def chunk(k, _):
        prej = prej_ref[...]
        pieces = []
        for u in range(CH):
            r = prei_ref[pl.ds(k * CH + u, 1), :]
            pieces.append(jnp.maximum(prej + r, 0.0).astype(jnp.bfloat16))
        hid = jnp.concatenate(pieces, axis=1)
        tmp = jnp.dot(hid, w2bd_ref[...],
                      preferred_element_type=jnp.float32)
        place = (l_iota == CH * k + u_iota).astype(jnp.bfloat16)
        gacc_ref[...] += jnp.dot(tmp.astype(jnp.bfloat16), place,
                                 preferred_element_type=jnp.float32)
        return 0

    jax.lax.fori_loop(0, NCH, chunk, 0)

    maskT = depT_ref[0] > 0
    T = jnp.where(maskT, gacc_ref[...], jnp.float32(-100.0))
    m = jnp.max(T, axis=0, keepdims=True)
    e = jnp.exp(T - m)
    betam = e / jnp.sum(e, axis=0, keepdims=True) * maskT.astype(jnp.float32)

    HjT = jnp.dot(wdT_ref[...], qT_ref[0],
                  preferred_element_type=jnp.float32)
    aggT = jnp.dot(HjT, betam, preferred_element_type=jnp.float32)
    outT_ref[0] = jnp.maximum(qgT_ref[0] + aggT, 0.0)


def kernel(queries, wordlens, syntactic_dep, W1, W2, Wd):
    q = queries.astype(jnp.float32)
    qT = jnp.swapaxes(q, 1, 2)
    depT = jnp.swapaxes(syntactic_dep.astype(jnp.int32), 1, 2)
    A = W1[:DIM, :]
    B = W1[DIM:, :]
    W2bd = jnp.kron(jnp.eye(CH, dtype=jnp.float32),
                    W2).astype(jnp.bfloat16)
    WdT = jnp.swapaxes(Wd, 0, 1)

    outT = pl.pallas_call(
        _gcn_body,
        grid=(BS, NG),
        in_specs=[
            pl.BlockSpec((1, L, DIM), lambda b, g: (b, 0, 0)),
            pl.BlockSpec((1, IG, DIM), lambda b, g: (b, g, 0)),
            pl.BlockSpec((1, DIM, L), lambda b, g: (b, 0, 0)),
            pl.BlockSpec((1, DIM, IG), lambda b, g: (b, 0, g)),
            pl.BlockSpec((1, L, IG), lambda b, g: (b, 0, g)),
            pl.BlockSpec((DIM, DIM), lambda b, g: (0, 0)),
            pl.BlockSpec((DIM, DIM), lambda b, g: (0, 0)),
            pl.BlockSpec((CH * DIM, CH), lambda b, g: (0, 0)),
            pl.BlockSpec((DIM, DIM), lambda b, g: (0, 0)),
        ],
        out_specs=pl.BlockSpec((1, DIM, IG), lambda b, g: (b, 0, g)),
        out_shape=jax.ShapeDtypeStruct((BS, DIM, L), jnp.float32),
        scratch_shapes=[
            pltpu.VMEM((IG, DIM), jnp.float32),
            pltpu.VMEM((L, DIM), jnp.float32),
            pltpu.VMEM((L, IG), jnp.float32),
        ],
        compiler_params=pltpu.CompilerParams(
            dimension_semantics=("arbitrary", "arbitrary"),
        ),
    )(q, q, qT, qT, depT, A, B, W2bd, WdT)

    out = jnp.swapaxes(outT, 1, 2)
    return (out, wordlens, syntactic_dep)


# speedup vs baseline: 1.7663x; 1.3017x over previous
import jax
import jax.numpy as jnp
from jax.experimental import pallas as pl
from jax.experimental.pallas import tpu as pltpu

BS, L, DIM = 4, 512, 128
IG = 128
NG = L // IG
CH = 16
NCH = IG // CH


def _gcn_body(q_ref, qg_ref, qT_ref, qgT_ref, depT_ref, a_ref, b_ref,
              w2bd_ref, wdT_ref, outT_ref, prei_ref, prej_ref,
              gacc_ref):
    prei_ref[...] = jnp.dot(qg_ref[0], a_ref[...],
                            preferred_element_type=jnp.float32)
    prej_ref[...] = jnp.dot(q_ref[0], b_ref[...],
                            preferred_element_type=jnp.float32)
    gacc_ref[...] = jnp.zeros((L, IG), jnp.float32)

    u_iota = jax.lax.broadcasted_iota(jnp.int32, (CH, IG), 0)
    l_iota = jax.lax.broadcasted_iota(jnp.int32, (CH, IG), 1)

    def chunk(k, _):
        prej = prej_ref[...]
        pieces = []
        for u in range(CH):
            r = prei_ref[pl.ds(k * CH + u, 1), :]
            pieces.append(jnp.maximum(prej + r, 0.0).astype(jnp.bfloat16))
        hid = jnp.concatenate(pieces, axis=1)
        tmp = jnp.dot(hid, w2bd_ref[...],
                      preferred_element_type=jnp.float32)
        place = (l_iota == CH * k + u_iota).astype(jnp.bfloat16)
        gacc_ref[...] += jnp.dot(tmp.astype(jnp.bfloat16), place,
                                 preferred_element_type=jnp.float32)
        return 0

    jax.lax.fori_loop(0, NCH, chunk, 0)

    maskT = depT_ref[0] > 0
    T = jnp.where(maskT, gacc_ref[...], jnp.float32(-100.0))
    m = jnp.max(T, axis=0, keepdims=True)
    e = jnp.exp(T - m)
    betam = e / jnp.sum(e, axis=0, keepdims=True) * maskT.astype(jnp.float32)

    HjT = jnp.dot(wdT_ref[...], qT_ref[0],
                  preferred_element_type=jnp.float32)
    aggT = jnp.dot(HjT, betam, preferred_element_type=jnp.float32)
    outT_ref[0] = jnp.maximum(qgT_ref[0] + aggT, 0.0)


def kernel(queries, wordlens, syntactic_dep, W1, W2, Wd):
    q = queries.astype(jnp.float32)
    qT = jnp.swapaxes(q, 1, 2)
    depT = jnp.swapaxes(syntactic_dep.astype(jnp.int32), 1, 2)
    A = W1[:DIM, :]
    B = W1[DIM:, :]
    W2bd = jnp.kron(jnp.eye(CH, dtype=jnp.float32),
                    W2).astype(jnp.bfloat16)
    WdT = jnp.swapaxes(Wd, 0, 1)

    outT = pl.pallas_call(
        _gcn_body,
        grid=(BS, NG),
        in_specs=[
            pl.BlockSpec((1, L, DIM), lambda b, g: (b, 0, 0)),
            pl.BlockSpec((1, IG, DIM), lambda b, g: (b, g, 0)),
            pl.BlockSpec((1, DIM, L), lambda b, g: (b, 0, 0)),
            pl.BlockSpec((1, DIM, IG), lambda b, g: (b, 0, g)),
            pl.BlockSpec((1, L, IG), lambda b, g: (b, 0, g)),
            pl.BlockSpec((DIM, DIM), lambda b, g: (0, 0)),
            pl.BlockSpec((DIM, DIM), lambda b, g: (0, 0)),
            pl.BlockSpec((CH * DIM, CH), lambda b, g: (0, 0)),
            pl.BlockSpec((DIM, DIM), lambda b, g: (0, 0)),
        ],
        out_specs=pl.BlockSpec((1, DIM, IG), lambda b, g: (b, 0, g)),
        out_shape=jax.ShapeDtypeStruct((BS, DIM, L), jnp.float32),
        scratch_shapes=[
            pltpu.VMEM((IG, DIM), jnp.float32),
            pltpu.VMEM((L, DIM), jnp.float32),
            pltpu.VMEM((L, IG), jnp.float32),
        ],
        compiler_params=pltpu.CompilerParams(
            dimension_semantics=("arbitrary", "arbitrary"),
        ),
    )(q, q, qT, qT, depT, A, B, W2bd, WdT)

    out = jnp.swapaxes(outT, 1, 2)
    return (out, wordlens, syntactic_dep)


# speedup vs baseline: 2.0953x; 1.1863x over previous
import jax
import jax.numpy as jnp
from jax.experimental import pallas as pl
from jax.experimental.pallas import tpu as pltpu

BS, L, DIM = 4, 512, 128
IG = 128
NG = L // IG
CH = 32
NCH = IG // CH


def _gcn_body(q_ref, qg_ref, qT_ref, qgT_ref, depT_ref, a_ref, b_ref,
              w2bd_ref, wdT_ref, outT_ref, prei_ref, prej_ref,
              gacc_ref):
    prei_ref[...] = jnp.dot(qg_ref[0], a_ref[...],
                            preferred_element_type=jnp.float32)
    prej_ref[...] = jnp.dot(q_ref[0], b_ref[...],
                            preferred_element_type=jnp.float32)
    gacc_ref[...] = jnp.zeros((L, IG), jnp.float32)

    u_iota = jax.lax.broadcasted_iota(jnp.int32, (CH, IG), 0)
    l_iota = jax.lax.broadcasted_iota(jnp.int32, (CH, IG), 1)

    def chunk(k, _):
        prej = prej_ref[...]
        pieces = []
        for u in range(CH):
            r = prei_ref[pl.ds(k * CH + u, 1), :]
            pieces.append(jnp.maximum(prej + r, 0.0).astype(jnp.bfloat16))
        hid = jnp.concatenate(pieces, axis=1)
        tmp = jnp.dot(hid, w2bd_ref[...],
                      preferred_element_type=jnp.float32)
        place = (l_iota == CH * k + u_iota).astype(jnp.bfloat16)
        gacc_ref[...] += jnp.dot(tmp.astype(jnp.bfloat16), place,
                                 preferred_element_type=jnp.float32)
        return 0

    jax.lax.fori_loop(0, NCH, chunk, 0)

    maskT = depT_ref[0] > 0
    T = jnp.where(maskT, gacc_ref[...], jnp.float32(-100.0))
    m = jnp.max(T, axis=0, keepdims=True)
    e = jnp.exp(T - m)
    betam = e / jnp.sum(e, axis=0, keepdims=True) * maskT.astype(jnp.float32)

    HjT = jnp.dot(wdT_ref[...], qT_ref[0],
                  preferred_element_type=jnp.float32)
    aggT = jnp.dot(HjT, betam, preferred_element_type=jnp.float32)
    outT_ref[0] = jnp.maximum(qgT_ref[0] + aggT, 0.0)


def kernel(queries, wordlens, syntactic_dep, W1, W2, Wd):
    q = queries.astype(jnp.float32)
    qT = jnp.swapaxes(q, 1, 2)
    depT = jnp.swapaxes(syntactic_dep.astype(jnp.int32), 1, 2)
    A = W1[:DIM, :]
    B = W1[DIM:, :]
    W2bd = jnp.kron(jnp.eye(CH, dtype=jnp.float32),
                    W2).astype(jnp.bfloat16)
    WdT = jnp.swapaxes(Wd, 0, 1)

    outT = pl.pallas_call(
        _gcn_body,
        grid=(BS, NG),
        in_specs=[
            pl.BlockSpec((1, L, DIM), lambda b, g: (b, 0, 0)),
            pl.BlockSpec((1, IG, DIM), lambda b, g: (b, g, 0)),
            pl.BlockSpec((1, DIM, L), lambda b, g: (b, 0, 0)),
            pl.BlockSpec((1, DIM, IG), lambda b, g: (b, 0, g)),
            pl.BlockSpec((1, L, IG), lambda b, g: (b, 0, g)),
            pl.BlockSpec((DIM, DIM), lambda b, g: (0, 0)),
            pl.BlockSpec((DIM, DIM), lambda b, g: (0, 0)),
            pl.BlockSpec((CH * DIM, CH), lambda b, g: (0, 0)),
            pl.BlockSpec((DIM, DIM), lambda b, g: (0, 0)),
        ],
        out_specs=pl.BlockSpec((1, DIM, IG), lambda b, g: (b, 0, g)),
        out_shape=jax.ShapeDtypeStruct((BS, DIM, L), jnp.float32),
        scratch_shapes=[
            pltpu.VMEM((IG, DIM), jnp.float32),
            pltpu.VMEM((L, DIM), jnp.float32),
            pltpu.VMEM((L, IG), jnp.float32),
        ],
        compiler_params=pltpu.CompilerParams(
            dimension_semantics=("arbitrary", "arbitrary"),
        ),
    )(q, q, qT, qT, depT, A, B, W2bd, WdT)

    out = jnp.swapaxes(outT, 1, 2)
    return (out, wordlens, syntactic_dep)


# speedup vs baseline: 2.4587x; 1.1734x over previous
import jax
import jax.numpy as jnp
from jax.experimental import pallas as pl
from jax.experimental.pallas import tpu as pltpu

BS, L, DIM = 4, 512, 128
IG = 128
NG = L // IG


def _gcn_body(q_ref, qg_ref, qT_ref, qgT_ref, depT_ref, a_ref, b_ref,
              w2bd_ref, wdT_ref, outT_ref):
    prei = jnp.dot(qg_ref[0], a_ref[...],
                   preferred_element_type=jnp.float32)
    prej = jnp.dot(q_ref[0], b_ref[...],
                   preferred_element_type=jnp.float32)

    pieces = [
        jnp.maximum(prej + prei[u:u + 1, :], 0.0).astype(jnp.bfloat16)
        for u in range(IG)
    ]
    hid = jnp.concatenate(pieces, axis=1)
    tT = jnp.dot(hid, w2bd_ref[...],
                 preferred_element_type=jnp.float32)

    maskT = depT_ref[0] > 0
    T = jnp.where(maskT, tT, jnp.float32(-100.0))
    m = jnp.max(T, axis=0, keepdims=True)
    e = jnp.exp(T - m)
    betam = e / jnp.sum(e, axis=0, keepdims=True) * maskT.astype(jnp.float32)

    HjT = jnp.dot(wdT_ref[...], qT_ref[0],
                  preferred_element_type=jnp.float32)
    aggT = jnp.dot(HjT, betam, preferred_element_type=jnp.float32)
    outT_ref[0] = jnp.maximum(qgT_ref[0] + aggT, 0.0)


def kernel(queries, wordlens, syntactic_dep, W1, W2, Wd):
    q = queries.astype(jnp.float32)
    qT = jnp.swapaxes(q, 1, 2)
    depT = jnp.swapaxes(syntactic_dep.astype(jnp.int32), 1, 2)
    A = W1[:DIM, :]
    B = W1[DIM:, :]
    W2bd = jnp.kron(jnp.eye(IG, dtype=jnp.float32),
                    W2).astype(jnp.bfloat16)
    WdT = jnp.swapaxes(Wd, 0, 1)

    outT = pl.pallas_call(
        _gcn_body,
        grid=(BS, NG),
        in_specs=[
            pl.BlockSpec((1, L, DIM), lambda b, g: (b, 0, 0)),
            pl.BlockSpec((1, IG, DIM), lambda b, g: (b, g, 0)),
            pl.BlockSpec((1, DIM, L), lambda b, g: (b, 0, 0)),
            pl.BlockSpec((1, DIM, IG), lambda b, g: (b, 0, g)),
            pl.BlockSpec((1, L, IG), lambda b, g: (b, 0, g)),
            pl.BlockSpec((DIM, DIM), lambda b, g: (0, 0)),
            pl.BlockSpec((DIM, DIM), lambda b, g: (0, 0)),
            pl.BlockSpec((IG * DIM, IG), lambda b, g: (0, 0)),
            pl.BlockSpec((DIM, DIM), lambda b, g: (0, 0)),
        ],
        out_specs=pl.BlockSpec((1, DIM, IG), lambda b, g: (b, 0, g)),
        out_shape=jax.ShapeDtypeStruct((BS, DIM, L), jnp.float32),
        compiler_params=pltpu.CompilerParams(
            dimension_semantics=("arbitrary", "arbitrary"),
        ),
    )(q, q, qT, qT, depT, A, B, W2bd, WdT)

    out = jnp.swapaxes(outT, 1, 2)
    return (out, wordlens, syntactic_dep)


# speedup vs baseline: 2.4643x; 1.0023x over previous
import jax
import jax.numpy as jnp
from jax.experimental import pallas as pl
from jax.experimental.pallas import tpu as pltpu

BS, L, DIM = 4, 512, 128
IG = 128
NG = L // IG


def _gcn_body(q_ref, qg_ref, qT_ref, qgT_ref, depT_ref, a_ref, b_ref,
              w2bd_ref, wdT_ref, outT_ref):
    prei = jnp.dot(qg_ref[0], a_ref[...],
                   preferred_element_type=jnp.float32)
    prej = jnp.dot(q_ref[0], b_ref[...],
                   preferred_element_type=jnp.float32)

    prej_h = prej.astype(jnp.bfloat16)
    prei_h = prei.astype(jnp.bfloat16)
    zero_h = jnp.bfloat16(0.0)
    pieces = [
        jnp.maximum(prej_h + prei_h[u:u + 1, :], zero_h)
        for u in range(IG)
    ]
    hid = jnp.concatenate(pieces, axis=1)
    tT = jnp.dot(hid, w2bd_ref[...],
                 preferred_element_type=jnp.float32)

    maskT = depT_ref[0] > 0
    T = jnp.where(maskT, tT, jnp.float32(-100.0))
    m = jnp.max(T, axis=0, keepdims=True)
    e = jnp.exp(T - m)
    betam = e / jnp.sum(e, axis=0, keepdims=True) * maskT.astype(jnp.float32)

    HjT = jnp.dot(wdT_ref[...], qT_ref[0],
                  preferred_element_type=jnp.float32)
    aggT = jnp.dot(HjT, betam, preferred_element_type=jnp.float32)
    outT_ref[0] = jnp.maximum(qgT_ref[0] + aggT, 0.0)


def kernel(queries, wordlens, syntactic_dep, W1, W2, Wd):
    q = queries.astype(jnp.float32)
    qT = jnp.swapaxes(q, 1, 2)
    depT = jnp.swapaxes(syntactic_dep.astype(jnp.int32), 1, 2)
    A = W1[:DIM, :]
    B = W1[DIM:, :]
    W2bd = jnp.kron(jnp.eye(IG, dtype=jnp.float32),
                    W2).astype(jnp.bfloat16)
    WdT = jnp.swapaxes(Wd, 0, 1)

    outT = pl.pallas_call(
        _gcn_body,
        grid=(BS, NG),
        in_specs=[
            pl.BlockSpec((1, L, DIM), lambda b, g: (b, 0, 0)),
            pl.BlockSpec((1, IG, DIM), lambda b, g: (b, g, 0)),
            pl.BlockSpec((1, DIM, L), lambda b, g: (b, 0, 0)),
            pl.BlockSpec((1, DIM, IG), lambda b, g: (b, 0, g)),
            pl.BlockSpec((1, L, IG), lambda b, g: (b, 0, g)),
            pl.BlockSpec((DIM, DIM), lambda b, g: (0, 0)),
            pl.BlockSpec((DIM, DIM), lambda b, g: (0, 0)),
            pl.BlockSpec((IG * DIM, IG), lambda b, g: (0, 0)),
            pl.BlockSpec((DIM, DIM), lambda b, g: (0, 0)),
        ],
        out_specs=pl.BlockSpec((1, DIM, IG), lambda b, g: (b, 0, g)),
        out_shape=jax.ShapeDtypeStruct((BS, DIM, L), jnp.float32),
        compiler_params=pltpu.CompilerParams(
            dimension_semantics=("arbitrary", "arbitrary"),
        ),
    )(q, q, qT, qT, depT, A, B, W2bd, WdT)

    out = jnp.swapaxes(outT, 1, 2)
    return (out, wordlens, syntactic_dep)


# speedup vs baseline: 2.4650x; 1.0003x over previous
import jax
import jax.numpy as jnp
from jax.experimental import pallas as pl
from jax.experimental.pallas import tpu as pltpu

BS, L, DIM = 4, 512, 128
IG = 128
NG = L // IG


def _gcn_body(q_ref, qg_ref, qT_ref, qgT_ref, depT_ref, a_ref, b_ref,
              w2bd_ref, wdT_ref, outT_ref):
    prei = jnp.dot(qg_ref[0], a_ref[...],
                   preferred_element_type=jnp.float32)
    prej = jnp.dot(q_ref[0], b_ref[...],
                   preferred_element_type=jnp.float32)

    prej_h = prej.astype(jnp.bfloat16)
    prei_h = prei.astype(jnp.bfloat16)
    zero_h = jnp.bfloat16(0.0)
    pieces = [
        jnp.maximum(prej_h + prei_h[u:u + 1, :], zero_h)
        for u in range(IG)
    ]
    tT = jnp.zeros((L, IG), jnp.float32)
    for g2 in range(IG // 2):
        part = jnp.concatenate(pieces[2 * g2:2 * g2 + 2], axis=1)
        w2s = w2bd_ref[2 * DIM * g2:2 * DIM * (g2 + 1), :]
        tT = tT + jnp.dot(part, w2s, preferred_element_type=jnp.float32)

    maskT = depT_ref[0] > 0
    T = jnp.where(maskT, tT, jnp.float32(-100.0))
    m = jnp.max(T, axis=0, keepdims=True)
    e = jnp.exp(T - m)
    betam = e / jnp.sum(e, axis=0, keepdims=True) * maskT.astype(jnp.float32)

    HjT = jnp.dot(wdT_ref[...], qT_ref[0],
                  preferred_element_type=jnp.float32)
    aggT = jnp.dot(HjT, betam, preferred_element_type=jnp.float32)
    outT_ref[0] = jnp.maximum(qgT_ref[0] + aggT, 0.0)


def kernel(queries, wordlens, syntactic_dep, W1, W2, Wd):
    q = queries.astype(jnp.float32)
    qT = jnp.swapaxes(q, 1, 2)
    depT = jnp.swapaxes(syntactic_dep.astype(jnp.int32), 1, 2)
    A = W1[:DIM, :]
    B = W1[DIM:, :]
    W2bd = jnp.kron(jnp.eye(IG, dtype=jnp.float32),
                    W2).astype(jnp.bfloat16)
    WdT = jnp.swapaxes(Wd, 0, 1)

    outT = pl.pallas_call(
        _gcn_body,
        grid=(BS, NG),
        in_specs=[
            pl.BlockSpec((1, L, DIM), lambda b, g: (b, 0, 0)),
            pl.BlockSpec((1, IG, DIM), lambda b, g: (b, g, 0)),
            pl.BlockSpec((1, DIM, L), lambda b, g: (b, 0, 0)),
            pl.BlockSpec((1, DIM, IG), lambda b, g: (b, 0, g)),
            pl.BlockSpec((1, L, IG), lambda b, g: (b, 0, g)),
            pl.BlockSpec((DIM, DIM), lambda b, g: (0, 0)),
            pl.BlockSpec((DIM, DIM), lambda b, g: (0, 0)),
            pl.BlockSpec((IG * DIM, IG), lambda b, g: (0, 0)),
            pl.BlockSpec((DIM, DIM), lambda b, g: (0, 0)),
        ],
        out_specs=pl.BlockSpec((1, DIM, IG), lambda b, g: (b, 0, g)),
        out_shape=jax.ShapeDtypeStruct((BS, DIM, L), jnp.float32),
        compiler_params=pltpu.CompilerParams(
            dimension_semantics=("arbitrary", "arbitrary"),
        ),
    )(q, q, qT, qT, depT, A, B, W2bd, WdT)

    out = jnp.swapaxes(outT, 1, 2)
    return (out, wordlens, syntactic_dep)
